# baseline (speedup 1.0000x reference)
import functools

import jax
import jax.numpy as jnp
from jax.experimental import pallas as pl
from jax.experimental.pallas import tpu as pltpu

H = 32
VOCAB = 64
LANES = 128


def _tree_sum(xs):
    while len(xs) > 1:
        xs = [xs[i] + xs[i + 1] for i in range(0, len(xs), 2)]
    return xs[0]


def _mm_kernel(tok_ref, embT_ref, W1T_ref, b1_ref, W2T_ref, b2_ref,
               g_ref, be_ref, WrT_ref, br_ref, WoT_ref, bo_ref,
               out_ref, mt_ref, hid_ref, rd_ref, mq_ref,
               *, chunk, num_chunks):
    l = pl.program_id(1)
    n = chunk * LANES

    embT = embT_ref[...]
    z1 = jnp.maximum(
        jnp.dot(W1T_ref[...], embT, preferred_element_type=jnp.float32)
        + b1_ref[...], 0.0)
    ff = jnp.dot(W2T_ref[...], z1,
                 preferred_element_type=jnp.float32) + b2_ref[...]
    x = embT + ff
    mu = jnp.mean(x, axis=0, keepdims=True)
    var = jnp.mean((x - mu) * (x - mu), axis=0, keepdims=True)
    tableT = (x - mu) * jax.lax.rsqrt(var + 1e-5) * g_ref[...] + be_ref[...]
    tok = tok_ref[0, 0]
    iota = jax.lax.broadcasted_iota(jnp.int32, (VOCAB, n), 0)
    onehot = jnp.where(iota == tok, 1.0, 0.0)
    hT = jnp.dot(tableT, onehot, preferred_element_type=jnp.float32)
    hid_ref[...] = hT
    d = jnp.sum(hT * hT, axis=0, keepdims=True) + 1e-6
    lane = jax.lax.broadcasted_iota(jnp.int32, (1, n), 1)
    is_q = jnp.logical_and(l == num_chunks - 1, lane >= n - LANES)
    mq = jnp.where(is_q, 0.0, 1.0)
    mq_ref[...] = mq
    rd_ref[...] = mq / d

    @pl.when(l == 0)
    def _init():
        mt_ref[...] = jnp.zeros_like(mt_ref)

    U = 8

    def minichunk(m, carry):
        t0 = U * m
        ks = [hid_ref[:, pl.ds((t0 + u) * LANES, LANES)] for u in range(U)]
        accs = [[None, None] for _ in range(U)]
        for j in range(H):
            mj = mt_ref[j]
            for u in range(U):
                p = mj * ks[u][j:j + 1, :]
                a = j & 1
                accs[u][a] = p if accs[u][a] is None else accs[u][a] + p
        deltas = []
        for u in range(U):
            w = accs[u][0] + accs[u][1]
            for s in range(u):
                g = jnp.sum(ks[s] * ks[u], axis=0, keepdims=True)
                w = w + deltas[s] * g
            rd = rd_ref[:, pl.ds((t0 + u) * LANES, LANES)]
            mq = mq_ref[:, pl.ds((t0 + u) * LANES, LANES)]
            deltas.append(ks[u] * mq - w * rd)
        for j in range(H):
            mj = mt_ref[j]
            for u in range(U):
                mj = mj + ks[u][j:j + 1, :] * deltas[u]
            mt_ref[j] = mj
        return carry

    jax.lax.fori_loop(0, chunk // U, minichunk, 0, unroll=False)

    @pl.when(l == num_chunks - 1)
    def _readout():
        q = hid_ref[:, pl.ds((chunk - 1) * LANES, LANES)]
        MT = mt_ref[...]
        ctx = jnp.sum(MT * q[:, None, :], axis=0)
        y = jnp.dot(WrT_ref[...], ctx,
                    preferred_element_type=jnp.float32) + br_ref[...]
        out_ref[...] = jnp.dot(WoT_ref[...], y,
                               preferred_element_type=jnp.float32) + bo_ref[...]


@jax.jit
def kernel(seq, embed, W1, b1, W2, b2, gamma, beta, Wr, br, Wo, bo):
    B, L = seq.shape
    chunk = 128
    num_chunks = L // chunk
    nb = B // LANES
    n = chunk * LANES

    tok = seq.astype(jnp.int32).reshape(nb, LANES, num_chunks, chunk)
    tok = tok.transpose(0, 2, 3, 1).reshape(nb, num_chunks, 1, n)

    col = lambda v: v.reshape(-1, 1)
    wspec = lambda shape: pl.BlockSpec(shape, lambda i, j: (0, 0))

    out = pl.pallas_call(
        functools.partial(_mm_kernel, chunk=chunk, num_chunks=num_chunks),
        grid=(nb, num_chunks),
        in_specs=[
            pl.BlockSpec((1, 1, 1, n), lambda i, j: (i, j, 0, 0)),
            wspec((H, VOCAB)),
            wspec((2 * H, H)),
            wspec((2 * H, 1)),
            wspec((H, 2 * H)),
            wspec((H, 1)),
            wspec((H, 1)),
            wspec((H, 1)),
            wspec((H, H)),
            wspec((H, 1)),
            wspec((VOCAB, H)),
            wspec((VOCAB, 1)),
        ],
        out_specs=pl.BlockSpec((VOCAB, LANES), lambda i, j: (0, i)),
        out_shape=jax.ShapeDtypeStruct((VOCAB, B), jnp.float32),
        scratch_shapes=[
            pltpu.VMEM((H, H, LANES), jnp.float32),
            pltpu.VMEM((H, n), jnp.float32),
            pltpu.VMEM((1, n), jnp.float32),
            pltpu.VMEM((1, n), jnp.float32),
        ],
        compiler_params=pltpu.CompilerParams(
            dimension_semantics=("parallel", "arbitrary"),
        ),
    )(tok, embed.T, W1.T, col(b1), W2.T, col(b2), col(gamma), col(beta),
      Wr.T, col(br), Wo.T, col(bo))
    return out.T


# speedup vs baseline: 1.0594x; 1.0594x over previous
import functools

import jax
import jax.numpy as jnp
from jax.experimental import pallas as pl
from jax.experimental.pallas import tpu as pltpu

H = 32
VOCAB = 64
LANES = 128


def _tree_sum(xs):
    while len(xs) > 1:
        xs = [xs[i] + xs[i + 1] for i in range(0, len(xs), 2)]
    return xs[0]


def _mm_kernel(tok_ref, embT_ref, W1T_ref, b1_ref, W2T_ref, b2_ref,
               g_ref, be_ref, WrT_ref, br_ref, WoT_ref, bo_ref,
               out_ref, mt_ref, hid_ref, rd_ref, mq_ref,
               *, chunk, num_chunks):
    l = pl.program_id(1)
    n = chunk * LANES

    embT = embT_ref[...]
    z1 = jnp.maximum(
        jnp.dot(W1T_ref[...], embT, preferred_element_type=jnp.float32)
        + b1_ref[...], 0.0)
    ff = jnp.dot(W2T_ref[...], z1,
                 preferred_element_type=jnp.float32) + b2_ref[...]
    x = embT + ff
    mu = jnp.mean(x, axis=0, keepdims=True)
    var = jnp.mean((x - mu) * (x - mu), axis=0, keepdims=True)
    tableT = (x - mu) * jax.lax.rsqrt(var + 1e-5) * g_ref[...] + be_ref[...]
    tok = tok_ref[0, 0]
    iota = jax.lax.broadcasted_iota(jnp.int32, (VOCAB, n), 0)
    onehot = jnp.where(iota == tok, 1.0, 0.0)
    hT = jnp.dot(tableT, onehot, preferred_element_type=jnp.float32)
    hid_ref[...] = hT
    d = jnp.sum(hT * hT, axis=0, keepdims=True) + 1e-6
    lane = jax.lax.broadcasted_iota(jnp.int32, (1, n), 1)
    is_q = jnp.logical_and(l == num_chunks - 1, lane >= n - LANES)
    mq = jnp.where(is_q, 0.0, 1.0)
    mq_ref[...] = mq
    rd_ref[...] = mq / d

    @pl.when(l == 0)
    def _init():
        mt_ref[...] = jnp.zeros_like(mt_ref)

    U = 4

    def minichunk(m, carry):
        t0 = U * m
        ks = [hid_ref[:, pl.ds((t0 + u) * LANES, LANES)] for u in range(U)]
        accs = [[None, None] for _ in range(U)]
        for j in range(H):
            mj = mt_ref[j]
            for u in range(U):
                p = mj * ks[u][j:j + 1, :]
                a = j & 1
                accs[u][a] = p if accs[u][a] is None else accs[u][a] + p
        deltas = []
        for u in range(U):
            w = accs[u][0] + accs[u][1]
            for s in range(u):
                g = jnp.sum(ks[s] * ks[u], axis=0, keepdims=True)
                w = w + deltas[s] * g
            rd = rd_ref[:, pl.ds((t0 + u) * LANES, LANES)]
            mq = mq_ref[:, pl.ds((t0 + u) * LANES, LANES)]
            deltas.append(ks[u] * mq - w * rd)
        for j in range(H):
            mj = mt_ref[j]
            for u in range(U):
                mj = mj + ks[u][j:j + 1, :] * deltas[u]
            mt_ref[j] = mj
        return carry

    def two(m, carry):
        minichunk(2 * m, carry)
        minichunk(2 * m + 1, carry)
        return carry

    jax.lax.fori_loop(0, chunk // (2 * U), two, 0, unroll=False)

    @pl.when(l == num_chunks - 1)
    def _readout():
        q = hid_ref[:, pl.ds((chunk - 1) * LANES, LANES)]
        MT = mt_ref[...]
        ctx = jnp.sum(MT * q[:, None, :], axis=0)
        y = jnp.dot(WrT_ref[...], ctx,
                    preferred_element_type=jnp.float32) + br_ref[...]
        out_ref[...] = jnp.dot(WoT_ref[...], y,
                               preferred_element_type=jnp.float32) + bo_ref[...]


@jax.jit
def kernel(seq, embed, W1, b1, W2, b2, gamma, beta, Wr, br, Wo, bo):
    B, L = seq.shape
    chunk = 128
    num_chunks = L // chunk
    nb = B // LANES
    n = chunk * LANES

    tok = seq.astype(jnp.int32).reshape(nb, LANES, num_chunks, chunk)
    tok = tok.transpose(0, 2, 3, 1).reshape(nb, num_chunks, 1, n)

    col = lambda v: v.reshape(-1, 1)
    wspec = lambda shape: pl.BlockSpec(shape, lambda i, j: (0, 0))

    out = pl.pallas_call(
        functools.partial(_mm_kernel, chunk=chunk, num_chunks=num_chunks),
        grid=(nb, num_chunks),
        in_specs=[
            pl.BlockSpec((1, 1, 1, n), lambda i, j: (i, j, 0, 0)),
            wspec((H, VOCAB)),
            wspec((2 * H, H)),
            wspec((2 * H, 1)),
            wspec((H, 2 * H)),
            wspec((H, 1)),
            wspec((H, 1)),
            wspec((H, 1)),
            wspec((H, H)),
            wspec((H, 1)),
            wspec((VOCAB, H)),
            wspec((VOCAB, 1)),
        ],
        out_specs=pl.BlockSpec((VOCAB, LANES), lambda i, j: (0, i)),
        out_shape=jax.ShapeDtypeStruct((VOCAB, B), jnp.float32),
        scratch_shapes=[
            pltpu.VMEM((H, H, LANES), jnp.float32),
            pltpu.VMEM((H, n), jnp.float32),
            pltpu.VMEM((1, n), jnp.float32),
            pltpu.VMEM((1, n), jnp.float32),
        ],
        compiler_params=pltpu.CompilerParams(
            dimension_semantics=("parallel", "arbitrary"),
        ),
    )(tok, embed.T, W1.T, col(b1), W2.T, col(b2), col(gamma), col(beta),
      Wr.T, col(br), Wo.T, col(bo))
    return out.T
